# baseline (speedup 1.0000x reference)
import functools

import jax
import jax.numpy as jnp
from jax import lax
from jax.experimental import pallas as pl
from jax.experimental.pallas import tpu as pltpu
from jax.experimental.pallas import tpu_sc as plsc

_N = 100000
_N_OPS = 1000
_ROW = 25
_L = 16
_TILE = 128

_info = plsc.get_sparse_core_info()
_NW = _info.num_cores * _info.num_subcores

_NFULL = (_N // _TILE) * _TILE
_NTAIL = _N - _NFULL
_NT = _NFULL // _TILE
_CHT = ((_NT + _NW - 1) // _NW + 1) // 2 * 2
_CH = _CHT * _TILE
_GROUPS = _CH // _L


def _fold_kernel(op_tableT_ref, op_mlp_W_ref, op_mlp_b_ref,
                 shape_WT_ref, shape_b_ref, shape_mlp_W_ref, shape_mlp_b_ref,
                 attr_WT_ref, attr_b_ref, attr_mlp_W_ref, attr_mlp_b_ref,
                 ft_ref, wvec_ref):
    ft = jnp.dot(op_mlp_W_ref[...], op_tableT_ref[...],
                 preferred_element_type=jnp.float32)
    ft_ref[...] = ft + op_mlp_b_ref[...][0]
    smlp = shape_mlp_W_ref[...]
    w_s = jnp.sum(shape_WT_ref[...] * smlp, axis=1)
    b_s = jnp.sum(smlp[0] * shape_b_ref[...]) + shape_mlp_b_ref[...]
    amlp = attr_mlp_W_ref[...]
    w_a = jnp.sum(attr_WT_ref[...] * amlp, axis=1)
    b_a = jnp.sum(amlp[0] * attr_b_ref[...]) + attr_mlp_b_ref[...]
    wvec_ref[...] = jnp.concatenate(
        [w_s, w_a, b_s, b_a, jnp.zeros((6,), jnp.float32)]).reshape(1, 32)


_mesh = plsc.VectorSubcoreMesh(core_axis_name="c", subcore_axis_name="s")


_STAGES = ((0, 8), (8, 6), (14, 6), (20, 6))
_SBUF = max(s for _, s in _STAGES) * _TILE


def _sc_body(geo_hbm, tail_hbm, ft_hbm, wv_hbm, out_hbm, tout_hbm,
             geo_v, geo_w, out_v, tg_v, tout_v, ft_v, wv_v, sem_a, sem_b):
    wid = lax.axis_index("s") * _info.num_cores + lax.axis_index("c")
    pltpu.sync_copy(wv_hbm, wv_v)
    pltpu.sync_copy(ft_hbm, ft_v)

    probe = plsc.load_gather(ft_v, [jnp.zeros((_L,), jnp.int32)])
    zdep = jnp.minimum(jnp.abs(probe).astype(jnp.int32), 0)
    wspl = [plsc.load_gather(wv_v, [jnp.full((_L,), c, jnp.int32) + zdep])
            for c in range(26)]

    def compute(src_v, dst_v, dst_off, ngroups):
        def body(g, carry):
            sl = pl.ds(g * _L, _L)
            op_idx = src_v[0, sl].astype(jnp.int32)
            op_val = plsc.load_gather(ft_v, [op_idx])
            acc_s = wspl[24]
            for c in range(8):
                acc_s = acc_s + src_v[1 + c, sl] * wspl[c]
            acc_a = wspl[25]
            for c in range(16):
                acc_a = acc_a + src_v[9 + c, sl] * wspl[8 + c]
            osl = pl.ds(dst_off + g * _L, _L)
            dst_v[0, osl] = jnp.abs(op_val)
            dst_v[1, osl] = jnp.abs(acc_s)
            dst_v[2, osl] = jnp.abs(acc_a)
            return carry
        lax.fori_loop(0, ngroups, body, jnp.int32(0))

    base = jnp.minimum(wid * _CHT, _NT - _CHT) * _TILE
    bufs = (geo_v, geo_w)
    sems = (sem_a, sem_b)

    def stage_copy(i):
        off, sz = _STAGES[i]
        return pltpu.async_copy(
            geo_hbm.at[:, pl.ds(base + off * _TILE, sz * _TILE)],
            bufs[i % 2].at[:, pl.ds(0, sz * _TILE)], sems[i % 2])

    pending = stage_copy(0)
    for i in range(len(_STAGES)):
        cur = pending
        if i + 1 < len(_STAGES):
            pending = stage_copy(i + 1)
        cur.wait()
        off, sz = _STAGES[i]
        compute(bufs[i % 2], out_v, off * _TILE, sz * _TILE // _L)
    pltpu.sync_copy(out_v, out_hbm.at[:, pl.ds(base, _CH)])

    @pl.when(wid == _NW - 1)
    def _():
        pltpu.sync_copy(tail_hbm, tg_v)
        compute(tg_v, tout_v, 0, _TILE // _L)
        pltpu.sync_copy(tout_v, tout_hbm)


def _make_sc_embed(interpret=False):
    return pl.kernel(
        _sc_body,
        out_type=(jax.ShapeDtypeStruct((3, _NFULL), jnp.float32),
                  jax.ShapeDtypeStruct((3, _TILE), jnp.float32)),
        mesh=_mesh,
        compiler_params=pltpu.CompilerParams(needs_layout_passes=False,
                                             use_tc_tiling_on_sc=True),
        scratch_types=[
            pltpu.VMEM((_ROW, _SBUF), jnp.float32),
            pltpu.VMEM((_ROW, _SBUF), jnp.float32),
            pltpu.VMEM((3, _CH), jnp.float32),
            pltpu.VMEM((_ROW, _TILE), jnp.float32),
            pltpu.VMEM((3, _TILE), jnp.float32),
            pltpu.VMEM((_N_OPS,), jnp.float32),
            pltpu.VMEM((32,), jnp.float32),
            pltpu.SemaphoreType.DMA,
            pltpu.SemaphoreType.DMA,
        ],
        interpret=interpret,
    )


_sc_embed = _make_sc_embed()


def kernel(geo_x, op_table, shape_W, shape_b, attr_W, attr_b,
           op_mlp_W, op_mlp_b, shape_mlp_W, shape_mlp_b,
           attr_mlp_W, attr_mlp_b):
    ft, wvec = pl.pallas_call(
        _fold_kernel,
        out_shape=(jax.ShapeDtypeStruct((1, _N_OPS), jnp.float32),
                   jax.ShapeDtypeStruct((1, 32), jnp.float32)),
    )(op_table.T, op_mlp_W, op_mlp_b, shape_W.T, shape_b, shape_mlp_W,
      shape_mlp_b, attr_W.T, attr_b, attr_mlp_W, attr_mlp_b)
    geoT = geo_x.T
    tail = jnp.pad(lax.slice(geoT, (0, _NFULL), (_ROW, _N)),
                   ((0, 0), (0, _TILE - _NTAIL)))
    out_full, out_tail = _sc_embed(geoT, tail, ft.reshape(-1),
                                   wvec.reshape(-1))
    out = jnp.concatenate([out_full, out_tail[:, :_NTAIL]], axis=1)
    return out.T
